# speedup vs baseline: 1.3230x; 1.3230x over previous
import functools

import jax
import jax.numpy as jnp
from jax import lax
from jax.experimental import pallas as pl
from jax.experimental.pallas import tpu as pltpu
from jax.experimental.pallas import tpu_sc as plsc

BATCH = 16384
NUM_FIELDS = 26
FIELD_DIM = 40000
TOTAL_VOCAB = NUM_FIELDS * FIELD_DIM
TDIM = 16

NC, NS, LANES = 2, 16, 16
NW = NC * NS
BPW = BATCH // NW
IDX_PER_W = BPW * NUM_FIELDS

_mesh = plsc.VectorSubcoreMesh(
    core_axis_name="c", subcore_axis_name="s", num_cores=NC, num_subcores=NS
)


@functools.partial(
    pl.kernel,
    out_type=jax.ShapeDtypeStruct((BATCH,), jnp.float32),
    mesh=_mesh,
    scratch_types=[
        pltpu.VMEM((IDX_PER_W,), jnp.int32),
        pltpu.VMEM((IDX_PER_W,), jnp.float32),
        pltpu.VMEM((TDIM * BPW,), jnp.float32),
        pltpu.VMEM((TDIM * LANES + LANES,), jnp.float32),
        pltpu.VMEM((BPW,), jnp.float32),
        pltpu.SemaphoreType.DMA,
    ],
)
def _fl_kernel(xw_hbm, tb_hbm, tab_hbm, pv_hbm, out_hbm,
               idx_v, vals_v, tb_v, pv_v, out_v, sem):
    wid = lax.axis_index("s") * NC + lax.axis_index("c")
    pltpu.sync_copy(xw_hbm.at[wid], idx_v)
    pltpu.sync_copy(tb_hbm.at[wid], tb_v)
    pltpu.sync_copy(pv_hbm, pv_v)
    pltpu.async_copy(tab_hbm.at[idx_v], vals_v, sem).wait()
    c0 = pv_v[pl.ds(TDIM * LANES, LANES)]
    for j in range(BPW // LANES):
        acc = c0
        for f in range(NUM_FIELDS):
            acc = acc + vals_v[pl.ds(f * BPW + j * LANES, LANES)]
        for k in range(TDIM):
            acc = acc + pv_v[pl.ds(k * LANES, LANES)] * tb_v[pl.ds(k * BPW + j * LANES, LANES)]
        out_v[pl.ds(j * LANES, LANES)] = acc
    pltpu.sync_copy(out_v, out_hbm.at[pl.ds(wid * BPW, BPW)])


def kernel(x, t, fc_table, lin_W, lin_b, bias):
    offsets = jnp.arange(NUM_FIELDS, dtype=x.dtype) * FIELD_DIM
    xi = x + offsets[None, :]
    xw = xi.reshape(NW, BPW, NUM_FIELDS).transpose(0, 2, 1).reshape(NW, IDX_PER_W)
    tb = t.reshape(NW, BPW, TDIM).transpose(0, 2, 1).reshape(NW, TDIM * BPW)
    tab = fc_table.reshape(TOTAL_VOCAB)
    pv = jnp.concatenate([
        jnp.repeat(lin_W.reshape(TDIM), LANES),
        jnp.broadcast_to((lin_b + bias).reshape(1), (LANES,)),
    ]).astype(jnp.float32)
    out = _fl_kernel(xw, tb, tab, pv)
    return out.reshape(BATCH, 1)


# speedup vs baseline: 1.4409x; 1.0891x over previous
import functools

import jax
import jax.numpy as jnp
from jax import lax
from jax.experimental import pallas as pl
from jax.experimental.pallas import tpu as pltpu
from jax.experimental.pallas import tpu_sc as plsc

BATCH = 16384
NUM_FIELDS = 26
FIELD_DIM = 40000
TOTAL_VOCAB = NUM_FIELDS * FIELD_DIM
TDIM = 16

NC, NS, LANES = 2, 16, 16
NW = NC * NS
BPW = BATCH // NW
IDX_PER_W = BPW * NUM_FIELDS
NSTAGE = 13
TSLICE = TOTAL_VOCAB // NSTAGE

_mesh = plsc.VectorSubcoreMesh(
    core_axis_name="c", subcore_axis_name="s", num_cores=NC, num_subcores=NS
)


@functools.partial(
    pl.kernel,
    out_type=jax.ShapeDtypeStruct((BATCH,), jnp.float32),
    mesh=_mesh,
    compiler_params=pltpu.CompilerParams(use_tc_tiling_on_sc=True),
    scratch_types=[
        pltpu.VMEM((IDX_PER_W,), jnp.int32),
        pltpu.VMEM((IDX_PER_W,), jnp.float32),
        pltpu.VMEM((TDIM * BPW,), jnp.float32),
        pltpu.VMEM((TDIM * LANES + LANES,), jnp.float32),
        pltpu.VMEM((BPW,), jnp.float32),
        pltpu.VMEM_SHARED((TOTAL_VOCAB,), jnp.float32),
        pltpu.SemaphoreType.DMA,
    ],
)
def _fl_kernel(xw_hbm, tb_hbm, tab_hbm, pv_hbm, out_hbm,
               idx_v, vals_v, tb_v, pv_v, out_v, tab_sh, sem):
    c = lax.axis_index("c")
    s = lax.axis_index("s")
    wid = s * NC + c

    @pl.when(s < NSTAGE)
    def _stage():
        pltpu.sync_copy(tab_hbm.at[pl.ds(s * TSLICE, TSLICE)],
                        tab_sh.at[pl.ds(s * TSLICE, TSLICE)])

    pltpu.sync_copy(xw_hbm.at[pl.ds(wid * IDX_PER_W, IDX_PER_W)], idx_v)
    pltpu.sync_copy(tb_hbm.at[pl.ds(wid * TDIM * BPW, TDIM * BPW)], tb_v)
    pltpu.sync_copy(pv_hbm, pv_v)

    plsc.subcore_barrier()

    pltpu.async_copy(tab_sh.at[idx_v], vals_v, sem).wait()

    c0 = pv_v[pl.ds(TDIM * LANES, LANES)]
    for j in range(BPW // LANES):
        acc = c0
        for f in range(NUM_FIELDS):
            acc = acc + vals_v[pl.ds(f * BPW + j * LANES, LANES)]
        for k in range(TDIM):
            acc = acc + pv_v[pl.ds(k * LANES, LANES)] * tb_v[pl.ds(k * BPW + j * LANES, LANES)]
        out_v[pl.ds(j * LANES, LANES)] = acc
    pltpu.sync_copy(out_v, out_hbm.at[pl.ds(wid * BPW, BPW)])


def kernel(x, t, fc_table, lin_W, lin_b, bias):
    offsets = jnp.arange(NUM_FIELDS, dtype=x.dtype) * FIELD_DIM
    xi = x + offsets[None, :]
    xw = xi.reshape(NW, BPW, NUM_FIELDS).transpose(0, 2, 1).reshape(NW * IDX_PER_W)
    tb = t.reshape(NW, BPW, TDIM).transpose(0, 2, 1).reshape(NW * TDIM * BPW)
    tab = fc_table.reshape(TOTAL_VOCAB)
    pv = jnp.concatenate([
        jnp.repeat(lin_W.reshape(TDIM), LANES),
        jnp.broadcast_to((lin_b + bias).reshape(1), (LANES,)),
    ]).astype(jnp.float32)
    out = _fl_kernel(xw, tb, tab, pv)
    return out.reshape(BATCH, 1)
